# baseline (speedup 1.0000x reference)
import functools

import jax
import jax.numpy as jnp
from jax import lax
from jax.experimental import pallas as pl
from jax.experimental.pallas import tpu as pltpu
from jax.experimental.pallas import tpu_sc as plsc

BATCH = 16384
EMBED_DIM = 32

_NC = 2
_NS = 16
_NW = _NC * _NS
_ROWS_PER_W = BATCH // _NW
_CHUNK = 64
_CHUNKS_PER_W = _ROWS_PER_W // _CHUNK
_L = 16


def _sc_gather(table, idx2d):
    mesh = plsc.VectorSubcoreMesh(core_axis_name="c", subcore_axis_name="s")

    @functools.partial(
        pl.kernel,
        mesh=mesh,
        compiler_params=pltpu.CompilerParams(needs_layout_passes=False),
        out_type=jax.ShapeDtypeStruct((BATCH, EMBED_DIM), jnp.float32),
        scratch_types=[
            pltpu.VMEM((_CHUNKS_PER_W, _CHUNK), jnp.int32),
            pltpu.VMEM((_CHUNK, EMBED_DIM), jnp.float32),
            pltpu.SemaphoreType.DMA,
        ],
    )
    def gather(table_hbm, idx_hbm, out_hbm, idx_v, rows_v, sem):
        wid = lax.axis_index("s") * _NC + lax.axis_index("c")
        lanes = lax.iota(jnp.int32, _L)
        pltpu.sync_copy(idx_hbm.at[pl.ds(wid * _CHUNKS_PER_W, _CHUNKS_PER_W)],
                        idx_v)
        for j in range(_CHUNKS_PER_W):
            copies = []
            for g in range(_CHUNK // _L):
                v16 = idx_v[j, pl.ds(g * _L, _L)]
                for t in range(_L):
                    r = jnp.sum(jnp.where(lanes == t, v16, 0))
                    copies.append(pltpu.async_copy(
                        table_hbm.at[pl.ds(r, 1)],
                        rows_v.at[pl.ds(g * _L + t, 1)], sem))
            for c in copies:
                c.wait()
            pltpu.sync_copy(
                rows_v,
                out_hbm.at[pl.ds(wid * _ROWS_PER_W + j * _CHUNK, _CHUNK)])

    return gather(table, idx2d)


_BB = 2048


def _dot(a, b):
    return jnp.dot(a, b, preferred_element_type=jnp.float32)


def _mlp_body(emb_ref, numT_ref, w1a_ref, w1b_ref, b1_ref, w2_ref, b2T_ref,
              outT_ref):
    h = _dot(emb_ref[...], w1a_ref[...])
    h = h + lax.dot_general(numT_ref[...], w1b_ref[...],
                            (((0,), (0,)), ((), ())),
                            preferred_element_type=jnp.float32)
    h = jnp.maximum(h + b1_ref[...], 0.0)
    outT_ref[...] = lax.dot_general(w2_ref[...], h,
                                    (((0,), (1,)), ((), ())),
                                    preferred_element_type=jnp.float32
                                    ) + b2T_ref[...]


def _tc_mlp(emb, numT, w1a, w1b, b1, w2, b2T):
    grid = (BATCH // _BB,)
    return pl.pallas_call(
        _mlp_body,
        grid=grid,
        in_specs=[
            pl.BlockSpec((_BB, EMBED_DIM), lambda i: (i, 0)),
            pl.BlockSpec((numT.shape[0], _BB), lambda i: (0, i)),
            pl.BlockSpec(w1a.shape, lambda i: (0, 0)),
            pl.BlockSpec(w1b.shape, lambda i: (0, 0)),
            pl.BlockSpec(b1.shape, lambda i: (0, 0)),
            pl.BlockSpec(w2.shape, lambda i: (0, 0)),
            pl.BlockSpec(b2T.shape, lambda i: (0, 0)),
        ],
        out_specs=pl.BlockSpec((EMBED_DIM, _BB), lambda i: (0, i)),
        out_shape=jax.ShapeDtypeStruct((EMBED_DIM, BATCH), jnp.float32),
    )(emb, numT, w1a, w1b, b1, w2, b2T)


def kernel(user_idx, numerical_features, user_embed, W1, b1, W2, b2):
    idx = user_idx.astype(jnp.int32)
    idx2d = idx.reshape(BATCH // _CHUNK, _CHUNK)
    emb = _sc_gather(user_embed, idx2d)
    outT = _tc_mlp(emb, numerical_features.T,
                   W1[:EMBED_DIM], W1[EMBED_DIM:],
                   b1.reshape(1, -1), W2, b2.reshape(-1, 1))
    return outT.T
